# baseline (speedup 1.0000x reference)
import jax
import jax.numpy as jnp
from jax import lax
from jax.experimental import pallas as pl
from jax.experimental.pallas import tpu as pltpu
from jax.experimental.pallas import tpu_sc as plsc

PAGE = 64
BPB = 64
NBUF = 8
DEPTH = 4


def _cache_body(parent_sm, child_sm, fresh_sm, seq_sm, pi_sm, cache_hbm,
                cache_out, bufs, srcpg_buf, in_sems, out_sems, srcpg_sem):
    parent = parent_sm[0]
    fresh = fresh_sm[0]
    src_len = seq_sm[parent]
    last_idx = jnp.maximum((src_len + PAGE - 1) // PAGE - 1, 0)
    has_partial = jnp.logical_and(src_len % PAGE != 0, src_len > 0)
    src_page = pi_sm[parent, last_idx]

    n_pages = cache_hbm.shape[0]
    nblk = n_pages // BPB

    srcpg_in = pltpu.make_async_copy(cache_hbm.at[pl.ds(src_page, 1)],
                                     srcpg_buf, srcpg_sem)
    srcpg_in.start()

    ins = [None] * nblk
    outs = [None] * nblk

    def start_in(i):
        b = i % NBUF
        c = pltpu.make_async_copy(cache_hbm.at[pl.ds(i * BPB, BPB)],
                                  bufs.at[b], in_sems.at[b])
        c.start()
        ins[i] = c

    def drain(j):
        b = j % NBUF
        ins[j].wait()
        blk_has_fresh = jnp.logical_and(
            has_partial,
            jnp.logical_and(fresh >= j * BPB, fresh < (j + 1) * BPB))

        @pl.when(blk_has_fresh)
        def _():
            bufs[b, pl.ds(fresh - j * BPB, 1)] = srcpg_buf[...]

        c = pltpu.make_async_copy(bufs.at[b], cache_out.at[pl.ds(j * BPB, BPB)],
                                  out_sems.at[b])
        c.start()
        outs[j] = c

    srcpg_in.wait()

    for i in range(nblk):
        if i >= NBUF:
            outs[i - NBUF].wait()
        start_in(i)
        j = i - DEPTH
        if j >= 0:
            drain(j)
    for j in range(max(nblk - DEPTH, 0), nblk):
        drain(j)
    for j in range(max(nblk - NBUF, 0), nblk):
        outs[j].wait()


def _cache_clone(parent, child, fresh, seq_lens, page_indices, cache):
    return pl.pallas_call(
        _cache_body,
        out_shape=jax.ShapeDtypeStruct(cache.shape, cache.dtype),
        in_specs=[
            pl.BlockSpec(memory_space=pltpu.SMEM),
            pl.BlockSpec(memory_space=pltpu.SMEM),
            pl.BlockSpec(memory_space=pltpu.SMEM),
            pl.BlockSpec(memory_space=pltpu.SMEM),
            pl.BlockSpec(memory_space=pltpu.SMEM),
            pl.BlockSpec(memory_space=pl.ANY),
        ],
        out_specs=pl.BlockSpec(memory_space=pl.ANY),
        scratch_shapes=[
            pltpu.VMEM((NBUF, BPB) + cache.shape[1:], cache.dtype),
            pltpu.VMEM((1,) + cache.shape[1:], cache.dtype),
            pltpu.SemaphoreType.DMA((NBUF,)),
            pltpu.SemaphoreType.DMA((NBUF,)),
            pltpu.SemaphoreType.DMA,
        ],
    )(parent, child, fresh, seq_lens, page_indices, cache)


def _state_clone(parent_a, child_a, fresh_a, tokens, seq_lens, page_indices):
    n_slots = tokens.shape[0]
    mesh = plsc.ScalarSubcoreMesh(axis_name="c", num_cores=2)
    rows_per_core = n_slots // 2

    @pl.kernel(
        out_type=(
            jax.ShapeDtypeStruct(tokens.shape, tokens.dtype),
            jax.ShapeDtypeStruct(seq_lens.shape, seq_lens.dtype),
            jax.ShapeDtypeStruct(page_indices.shape, page_indices.dtype),
        ),
        mesh=mesh,
        scratch_types=[
            pltpu.SMEM((1,), jnp.int32),
            pltpu.SMEM((1,), jnp.int32),
            pltpu.SMEM((1,), jnp.int32),
            pltpu.SMEM(seq_lens.shape, seq_lens.dtype),
            pltpu.SMEM((1, page_indices.shape[1]), page_indices.dtype),
            pltpu.VMEM_SHARED((rows_per_core, tokens.shape[1]), tokens.dtype),
            pltpu.VMEM_SHARED((rows_per_core, page_indices.shape[1]),
                              page_indices.dtype),
            pltpu.SemaphoreType.DMA,
            pltpu.SemaphoreType.DMA,
            pltpu.SemaphoreType.DMA,
        ],
    )
    def state_kernel(parent_hbm, child_hbm, fresh_hbm, tokens_hbm, seq_hbm,
                     pi_hbm, tok_out, seq_out, pi_out, parent_sm, child_sm,
                     fresh_sm, seq_sm, pirow_sm, tok_sp, pi_sp, sem, tok_sem,
                     pi_sem):
        core = lax.axis_index("c")
        base = core * rows_per_core

        tok_in = pltpu.make_async_copy(
            tokens_hbm.at[pl.ds(base, rows_per_core)], tok_sp, tok_sem)
        tok_in.start()
        pi_in = pltpu.make_async_copy(
            pi_hbm.at[pl.ds(base, rows_per_core)], pi_sp, pi_sem)
        pi_in.start()

        pltpu.async_copy(parent_hbm, parent_sm, sem).wait()
        pltpu.async_copy(child_hbm, child_sm, sem).wait()
        pltpu.async_copy(fresh_hbm, fresh_sm, sem).wait()
        pltpu.async_copy(seq_hbm, seq_sm, sem).wait()
        parent = parent_sm[0]
        child = child_sm[0]
        fresh = fresh_sm[0]
        src_len = seq_sm[parent]
        last_idx = jnp.maximum((src_len + PAGE - 1) // PAGE - 1, 0)
        has_partial = jnp.logical_and(src_len % PAGE != 0, src_len > 0)

        @pl.when(core == 0)
        def _():
            seq_sm[child] = src_len
            pltpu.async_copy(seq_sm, seq_out, sem).wait()

        tok_in.wait()
        pi_in.wait()

        owner = jnp.where(child >= rows_per_core, 1, 0)

        @pl.when(core == owner)
        def _():
            pltpu.async_copy(tokens_hbm.at[pl.ds(parent, 1)],
                             tok_sp.at[pl.ds(child - base, 1)], sem).wait()

        tok_w = pltpu.make_async_copy(
            tok_sp, tok_out.at[pl.ds(base, rows_per_core)], tok_sem)
        tok_w.start()
        pi_w = pltpu.make_async_copy(
            pi_sp, pi_out.at[pl.ds(base, rows_per_core)], pi_sem)
        pi_w.start()
        tok_w.wait()
        pi_w.wait()

        @pl.when(core == owner)
        def _():
            pltpu.async_copy(pi_hbm.at[pl.ds(parent, 1)], pirow_sm, sem).wait()

            @pl.when(has_partial)
            def _():
                pirow_sm[0, last_idx] = fresh

            pltpu.async_copy(pirow_sm, pi_out.at[pl.ds(child, 1)], sem).wait()

    return state_kernel(parent_a, child_a, fresh_a, tokens, seq_lens,
                        page_indices)


def kernel(tokens, seq_lens, page_indices, cache, parent_local_id,
           child_local_id, fresh_page):
    parent = jnp.asarray(parent_local_id, jnp.int32).reshape(1)
    child = jnp.asarray(child_local_id, jnp.int32).reshape(1)
    fresh = jnp.asarray(fresh_page, jnp.int32).reshape(1)
    cache_out = _cache_clone(parent, child, fresh, seq_lens, page_indices,
                             cache)
    tokens_out, seq_out, pi_out = _state_clone(parent, child, fresh, tokens,
                                               seq_lens, page_indices)
    return tokens_out, seq_out, pi_out, cache_out


# speedup vs baseline: 1.1550x; 1.1550x over previous
import jax
import jax.numpy as jnp
from jax import lax
from jax.experimental import pallas as pl
from jax.experimental.pallas import tpu as pltpu

PAGE = 64
BPB = 64
NBUF = 8
DEPTH = 4


def _clone_body(scal_ref, seq_sm, pi_sm, seq_in, pi_in, tokens_hbm, cache_hbm,
                seq_out, pi_out, tokens_out, cache_out,
                bufs, tok_buf, srcpg_buf, in_sems, out_sems, tok_sem,
                srcpg_sem):
    parent = scal_ref[0]
    child = scal_ref[1]
    fresh = scal_ref[2]
    src_len = seq_sm[parent]
    last_idx = jnp.maximum((src_len + PAGE - 1) // PAGE - 1, 0)
    has_partial = jnp.logical_and(src_len % PAGE != 0, src_len > 0)
    src_page = pi_sm[parent, last_idx]

    n_pages = cache_hbm.shape[0]
    nblk = n_pages // BPB

    tok_in = pltpu.make_async_copy(tokens_hbm, tok_buf, tok_sem)
    tok_in.start()
    srcpg_in = pltpu.make_async_copy(cache_hbm.at[pl.ds(src_page, 1)],
                                     srcpg_buf, srcpg_sem)
    srcpg_in.start()

    ins = [None] * nblk
    outs = [None] * nblk

    def start_in(i):
        b = i % NBUF
        c = pltpu.make_async_copy(cache_hbm.at[pl.ds(i * BPB, BPB)],
                                  bufs.at[b], in_sems.at[b])
        c.start()
        ins[i] = c

    def drain(j):
        b = j % NBUF
        ins[j].wait()
        blk_has_fresh = jnp.logical_and(
            has_partial,
            jnp.logical_and(fresh >= j * BPB, fresh < (j + 1) * BPB))

        @pl.when(blk_has_fresh)
        def _():
            bufs[b, pl.ds(fresh - j * BPB, 1)] = srcpg_buf[...]

        c = pltpu.make_async_copy(bufs.at[b], cache_out.at[pl.ds(j * BPB, BPB)],
                                  out_sems.at[b])
        c.start()
        outs[j] = c

    srcpg_in.wait()

    for i in range(nblk):
        if i >= NBUF:
            outs[i - NBUF].wait()
        start_in(i)
        j = i - DEPTH
        if j >= 0:
            drain(j)

    tok_in.wait()
    row = tok_buf[pl.ds(parent, 1), :]
    tok_buf[pl.ds(child, 1), :] = row
    tok_out = pltpu.make_async_copy(tok_buf, tokens_out, tok_sem)
    tok_out.start()

    n_slots = seq_in.shape[1]
    iota_slot = lax.broadcasted_iota(jnp.int32, (1, n_slots), 1)
    seq_out[...] = jnp.where(iota_slot == child, src_len, seq_in[...])

    pi_v = pi_in[...]
    nrow, ncol = pi_v.shape
    row_i = lax.broadcasted_iota(jnp.int32, (nrow, ncol), 0)
    col_i = lax.broadcasted_iota(jnp.int32, (1, ncol), 1)
    parent_row = jnp.sum(jnp.where(row_i == parent, pi_v, 0), axis=0,
                         keepdims=True)
    child_row = jnp.where(
        jnp.logical_and(col_i == last_idx, has_partial), fresh, parent_row)
    pi_out[...] = jnp.where(row_i == child, child_row, pi_v)

    for j in range(max(nblk - DEPTH, 0), nblk):
        drain(j)
    for j in range(max(nblk - NBUF, 0), nblk):
        outs[j].wait()
    tok_out.wait()


def kernel(tokens, seq_lens, page_indices, cache, parent_local_id,
           child_local_id, fresh_page):
    scal = jnp.stack([
        jnp.asarray(parent_local_id, jnp.int32),
        jnp.asarray(child_local_id, jnp.int32),
        jnp.asarray(fresh_page, jnp.int32),
    ])
    seq2d = seq_lens.reshape(1, -1)

    out_shapes = (
        jax.ShapeDtypeStruct(seq2d.shape, seq_lens.dtype),
        jax.ShapeDtypeStruct(page_indices.shape, page_indices.dtype),
        jax.ShapeDtypeStruct(tokens.shape, tokens.dtype),
        jax.ShapeDtypeStruct(cache.shape, cache.dtype),
    )
    seq_out, pi_out, tokens_out, cache_out = pl.pallas_call(
        _clone_body,
        out_shape=out_shapes,
        in_specs=[
            pl.BlockSpec(memory_space=pltpu.SMEM),
            pl.BlockSpec(memory_space=pltpu.SMEM),
            pl.BlockSpec(memory_space=pltpu.SMEM),
            pl.BlockSpec(memory_space=pltpu.VMEM),
            pl.BlockSpec(memory_space=pltpu.VMEM),
            pl.BlockSpec(memory_space=pl.ANY),
            pl.BlockSpec(memory_space=pl.ANY),
        ],
        out_specs=[
            pl.BlockSpec(memory_space=pltpu.VMEM),
            pl.BlockSpec(memory_space=pltpu.VMEM),
            pl.BlockSpec(memory_space=pl.ANY),
            pl.BlockSpec(memory_space=pl.ANY),
        ],
        scratch_shapes=[
            pltpu.VMEM((NBUF, BPB) + cache.shape[1:], cache.dtype),
            pltpu.VMEM(tokens.shape, tokens.dtype),
            pltpu.VMEM((1,) + cache.shape[1:], cache.dtype),
            pltpu.SemaphoreType.DMA((NBUF,)),
            pltpu.SemaphoreType.DMA((NBUF,)),
            pltpu.SemaphoreType.DMA,
            pltpu.SemaphoreType.DMA,
        ],
    )(scal, seq_lens, page_indices, seq2d, page_indices, tokens, cache)

    return tokens_out, seq_out.reshape(-1), pi_out, cache_out


# speedup vs baseline: 1.1593x; 1.0038x over previous
import jax
import jax.numpy as jnp
from jax import lax
from jax.experimental import pallas as pl
from jax.experimental.pallas import tpu as pltpu

PAGE = 64
BPB = 64
NBUF = 8
DEPTH = 4


def _clone_body(scal_ref, seq_sm, pi_sm, seq_in, pi_in, tokens_hbm, cache_hbm,
                seq_out, pi_out, tokens_out, cache_out,
                bufs, tok_buf, srcpg_buf, in_sems, out_sems, tok_sem,
                srcpg_sem):
    parent = scal_ref[0]
    child = scal_ref[1]
    fresh = scal_ref[2]
    src_len = seq_sm[parent]
    last_idx = jnp.maximum((src_len + PAGE - 1) // PAGE - 1, 0)
    has_partial = jnp.logical_and(src_len % PAGE != 0, src_len > 0)
    src_page = pi_sm[parent, last_idx]

    n_pages = cache_hbm.shape[0]
    nblk = n_pages // BPB

    tok_in = pltpu.make_async_copy(tokens_hbm, tok_buf, tok_sem)
    tok_in.start()
    srcpg_in = pltpu.make_async_copy(cache_hbm.at[pl.ds(src_page, 1)],
                                     srcpg_buf, srcpg_sem)
    srcpg_in.start()

    ins = [None] * nblk
    outs = [None] * nblk

    def start_in(i):
        b = i % NBUF
        c = pltpu.make_async_copy(cache_hbm.at[pl.ds(i * BPB, BPB)],
                                  bufs.at[b], in_sems.at[b])
        c.start()
        ins[i] = c

    def drain(j):
        b = j % NBUF
        if j == 0:
            srcpg_in.wait()
        ins[j].wait()
        blk_has_fresh = jnp.logical_and(
            has_partial,
            jnp.logical_and(fresh >= j * BPB, fresh < (j + 1) * BPB))

        @pl.when(blk_has_fresh)
        def _():
            bufs[b, pl.ds(fresh - j * BPB, 1)] = srcpg_buf[...]

        c = pltpu.make_async_copy(bufs.at[b], cache_out.at[pl.ds(j * BPB, BPB)],
                                  out_sems.at[b])
        c.start()
        outs[j] = c

    for i in range(nblk):
        if i >= NBUF:
            outs[i - NBUF].wait()
        start_in(i)
        j = i - DEPTH
        if j >= 0:
            drain(j)

    tok_in.wait()
    row = tok_buf[pl.ds(parent, 1), :]
    tok_buf[pl.ds(child, 1), :] = row
    tok_out = pltpu.make_async_copy(tok_buf, tokens_out, tok_sem)
    tok_out.start()

    n_slots = seq_in.shape[1]
    iota_slot = lax.broadcasted_iota(jnp.int32, (1, n_slots), 1)
    seq_out[...] = jnp.where(iota_slot == child, src_len, seq_in[...])

    pi_v = pi_in[...]
    nrow, ncol = pi_v.shape
    row_i = lax.broadcasted_iota(jnp.int32, (nrow, ncol), 0)
    col_i = lax.broadcasted_iota(jnp.int32, (1, ncol), 1)
    parent_row = jnp.sum(jnp.where(row_i == parent, pi_v, 0), axis=0,
                         keepdims=True)
    child_row = jnp.where(
        jnp.logical_and(col_i == last_idx, has_partial), fresh, parent_row)
    pi_out[...] = jnp.where(row_i == child, child_row, pi_v)

    for j in range(max(nblk - DEPTH, 0), nblk):
        drain(j)
    for j in range(max(nblk - NBUF, 0), nblk):
        outs[j].wait()
    tok_out.wait()


def kernel(tokens, seq_lens, page_indices, cache, parent_local_id,
           child_local_id, fresh_page):
    scal = jnp.stack([
        jnp.asarray(parent_local_id, jnp.int32),
        jnp.asarray(child_local_id, jnp.int32),
        jnp.asarray(fresh_page, jnp.int32),
    ])
    seq2d = seq_lens.reshape(1, -1)

    out_shapes = (
        jax.ShapeDtypeStruct(seq2d.shape, seq_lens.dtype),
        jax.ShapeDtypeStruct(page_indices.shape, page_indices.dtype),
        jax.ShapeDtypeStruct(tokens.shape, tokens.dtype),
        jax.ShapeDtypeStruct(cache.shape, cache.dtype),
    )
    seq_out, pi_out, tokens_out, cache_out = pl.pallas_call(
        _clone_body,
        out_shape=out_shapes,
        in_specs=[
            pl.BlockSpec(memory_space=pltpu.SMEM),
            pl.BlockSpec(memory_space=pltpu.SMEM),
            pl.BlockSpec(memory_space=pltpu.SMEM),
            pl.BlockSpec(memory_space=pltpu.VMEM),
            pl.BlockSpec(memory_space=pltpu.VMEM),
            pl.BlockSpec(memory_space=pl.ANY),
            pl.BlockSpec(memory_space=pl.ANY),
        ],
        out_specs=[
            pl.BlockSpec(memory_space=pltpu.VMEM),
            pl.BlockSpec(memory_space=pltpu.VMEM),
            pl.BlockSpec(memory_space=pl.ANY),
            pl.BlockSpec(memory_space=pl.ANY),
        ],
        scratch_shapes=[
            pltpu.VMEM((NBUF, BPB) + cache.shape[1:], cache.dtype),
            pltpu.VMEM(tokens.shape, tokens.dtype),
            pltpu.VMEM((1,) + cache.shape[1:], cache.dtype),
            pltpu.SemaphoreType.DMA((NBUF,)),
            pltpu.SemaphoreType.DMA((NBUF,)),
            pltpu.SemaphoreType.DMA,
            pltpu.SemaphoreType.DMA,
        ],
    )(scal, seq_lens, page_indices, seq2d, page_indices, tokens, cache)

    return tokens_out, seq_out.reshape(-1), pi_out, cache_out
